# baseline (speedup 1.0000x reference)
import functools

import jax
import jax.numpy as jnp
from jax.experimental import pallas as pl
from jax.experimental.pallas import tpu as pltpu
from jax.experimental.pallas import tpu_sc as plsc

_Q = 1024
_D = 16
_K = 5
_C = 4096


def _knn_body(x2_ref, xt_ref, ma_ref, mo_ref, tq_ref, out_ref,
              s1_ref, s2_ref, s3_ref, s4_ref, *, nsteps):
    j = pl.program_id(0)
    inf = jnp.float32(jnp.inf)

    @pl.when(j == 0)
    def _init():
        full = jnp.full((_Q, 128), jnp.inf, dtype=jnp.float32)
        s1_ref[...] = full
        s2_ref[...] = full
        s3_ref[...] = full
        s4_ref[...] = full

    x2 = x2_ref[...]
    xt = xt_ref[...]
    m_and = ma_ref[...]
    m_or = mo_ref[...]
    tq = tq_ref[...]

    cross2 = jax.lax.dot_general(
        x2, xt, dimension_numbers=(((1,), (1,)), ((), ())),
        preferred_element_type=jnp.float32)
    xsq = 0.25 * jnp.sum(x2 * x2, axis=1, keepdims=True)
    d2 = (cross2 + xsq) + tq

    ki = jax.lax.bitcast_convert_type(d2, jnp.int32)
    ki = jnp.bitwise_or(jnp.bitwise_and(ki, m_and), m_or)
    keys = jax.lax.bitcast_convert_type(ki, jnp.float32)

    h = _C // 2
    a, b = keys[:, :h], keys[:, h:]
    m1 = jnp.minimum(a, b)
    m2 = jnp.maximum(a, b)
    while h > 128:
        h //= 2
        a1, b1 = m1[:, :h], m1[:, h:]
        a2, b2 = m2[:, :h], m2[:, h:]
        m2 = jnp.minimum(jnp.maximum(a1, b1), jnp.minimum(a2, b2))
        m1 = jnp.minimum(a1, b1)

    a1, a2, a3, a4 = s1_ref[...], s2_ref[...], s3_ref[...], s4_ref[...]
    c1 = jnp.minimum(a1, m1)
    c2 = jnp.minimum(jnp.minimum(a2, jnp.maximum(a1, m1)), m2)
    c3 = jnp.minimum(a3, jnp.minimum(jnp.maximum(a2, m1), jnp.maximum(a1, m2)))
    c4 = jnp.minimum(a4, jnp.minimum(jnp.maximum(a3, m1), jnp.maximum(a2, m2)))
    s1_ref[...] = c1
    s2_ref[...] = c2
    s3_ref[...] = c3
    s4_ref[...] = c4

    @pl.when(j == nsteps - 1)
    def _finish():
        out_ref[...] = jnp.concatenate([c1, c2, c3, c4], axis=1)


_NC = 2
_NS = 16
_NW = _NC * _NS
_QPW = _Q // _NW
_CAND = 512


def _sc_vote_body(cand_hbm, out_hbm, slab_v, out_v):
    wid = jax.lax.axis_index("s") * _NC + jax.lax.axis_index("c")
    base = wid * _QPW
    pltpu.sync_copy(cand_hbm.at[pl.ds(wid * _QPW * _CAND, _QPW * _CAND)],
                    slab_v)
    inf = jnp.full((16,), jnp.inf, dtype=jnp.float32)

    def rbody(r, carry):
        new = []
        for h in range(2):
            ts = carry[h * _K:(h + 1) * _K]
            a = slab_v[pl.ds(r * _QPW + h * 16, 16)]
            for t in ts:
                new.append(jnp.minimum(t, a))
                a = jnp.maximum(t, a)
        return tuple(new)

    tops = jax.lax.fori_loop(0, _CAND, rbody, (inf,) * (2 * _K))
    for h in range(2):
        for i, t in enumerate(tops[h * _K:(h + 1) * _K]):
            out_v[pl.ds((i * 2 + h) * 16, 16)] = t
    pltpu.sync_copy(out_v, out_hbm.at[pl.ds(base * _K, _QPW * _K)])


def _sc_merge(cand_flat):
    mesh = plsc.VectorSubcoreMesh(core_axis_name="c", subcore_axis_name="s")
    f = functools.partial(
        pl.kernel,
        mesh=mesh,
        out_type=jax.ShapeDtypeStruct((_Q * _K,), jnp.float32),
        scratch_types=[
            pltpu.VMEM((_QPW * _CAND,), jnp.float32),
            pltpu.VMEM((_QPW * _K,), jnp.float32),
        ],
    )(_sc_vote_body)
    return f(cand_flat)


@jax.jit
def kernel(x, X_train, y_train):
    n = X_train.shape[0]
    nc = (n + _C - 1) // _C
    npad = nc * _C - n
    m_and = jnp.pad(jnp.full((1, n), -2, dtype=jnp.int32), ((0, 0), (0, npad)))
    m_or = jnp.pad(y_train[None, :], ((0, 0), (0, npad)),
                   constant_values=0x7F7FFFFF)
    tqp = jnp.pad(jnp.sum(X_train * X_train, axis=1)[None, :],
                  ((0, 0), (0, npad)))
    x2 = x.reshape(_Q, _D) * jnp.float32(-2.0)

    out = pl.pallas_call(
        functools.partial(_knn_body, nsteps=nc),
        grid=(nc,),
        in_specs=[
            pl.BlockSpec((_Q, _D), lambda j: (0, 0)),
            pl.BlockSpec((_C, _D), lambda j: (j, 0)),
            pl.BlockSpec((1, _C), lambda j: (0, j)),
            pl.BlockSpec((1, _C), lambda j: (0, j)),
            pl.BlockSpec((1, _C), lambda j: (0, j)),
        ],
        out_specs=pl.BlockSpec((_Q, 512), lambda j: (0, 0)),
        out_shape=jax.ShapeDtypeStruct((_Q, 512), jnp.float32),
        scratch_shapes=[pltpu.VMEM((_Q, 128), jnp.float32)] * 4,
    )(x2, X_train, m_and, m_or, tqp)
    candw = out.reshape(_NW, _QPW, 512).transpose(0, 2, 1)
    top5 = _sc_merge(candw.reshape(-1))
    bits = jnp.bitwise_and(
        jax.lax.bitcast_convert_type(top5, jnp.int32), jnp.int32(1))
    votes = jnp.sum(bits.reshape(_NW, _K, _QPW), axis=1).reshape(_Q, 1)
    return (votes > _K // 2).astype(jnp.float32)


# speedup vs baseline: 1.0021x; 1.0021x over previous
import functools

import jax
import jax.numpy as jnp
from jax.experimental import pallas as pl
from jax.experimental.pallas import tpu as pltpu
from jax.experimental.pallas import tpu_sc as plsc

_Q = 1024
_D = 16
_K = 5
_C = 4096


def _knn_body(x2_ref, xt_ref, ma_ref, mo_ref, tq_ref, out_ref,
              s1_ref, s2_ref, s3_ref, s4_ref, *, nsteps):
    j = pl.program_id(0)
    inf = jnp.float32(jnp.inf)

    @pl.when(j == 0)
    def _init():
        full = jnp.full((_Q, 128), jnp.inf, dtype=jnp.float32)
        s1_ref[...] = full
        s2_ref[...] = full
        s3_ref[...] = full
        s4_ref[...] = full

    x2 = x2_ref[...]
    xt = xt_ref[...]
    m_and = ma_ref[...]
    m_or = mo_ref[...]
    tq = tq_ref[...]

    cross2 = jax.lax.dot_general(
        x2, xt, dimension_numbers=(((1,), (1,)), ((), ())),
        preferred_element_type=jnp.float32)
    xsq = 0.25 * jnp.sum(x2 * x2, axis=1, keepdims=True)
    d2 = (cross2 + xsq) + tq

    ki = jax.lax.bitcast_convert_type(d2, jnp.int32)
    ki = jnp.bitwise_or(jnp.bitwise_and(ki, m_and), m_or)
    keys = jax.lax.bitcast_convert_type(ki, jnp.float32)

    h = _C // 2
    a, b = keys[:, :h], keys[:, h:]
    m1 = jnp.minimum(a, b)
    m2 = jnp.maximum(a, b)
    while h > 128:
        h //= 2
        a1, b1 = m1[:, :h], m1[:, h:]
        a2, b2 = m2[:, :h], m2[:, h:]
        m2 = jnp.minimum(jnp.maximum(a1, b1), jnp.minimum(a2, b2))
        m1 = jnp.minimum(a1, b1)

    a1, a2, a3, a4 = s1_ref[...], s2_ref[...], s3_ref[...], s4_ref[...]
    c1 = jnp.minimum(a1, m1)
    c2 = jnp.minimum(jnp.minimum(a2, jnp.maximum(a1, m1)), m2)
    c3 = jnp.minimum(a3, jnp.minimum(jnp.maximum(a2, m1), jnp.maximum(a1, m2)))
    c4 = jnp.minimum(a4, jnp.minimum(jnp.maximum(a3, m1), jnp.maximum(a2, m2)))
    s1_ref[...] = c1
    s2_ref[...] = c2
    s3_ref[...] = c3
    s4_ref[...] = c4

    @pl.when(j == nsteps - 1)
    def _finish():
        out_ref[...] = jnp.concatenate([c1, c2, c3, c4], axis=1)


_NC = 2
_NS = 16
_NW = _NC * _NS
_QPW = _Q // _NW
_CAND = 512


def _sc_vote_body(cand_hbm, out_hbm, slab_v, out_v):
    wid = jax.lax.axis_index("s") * _NC + jax.lax.axis_index("c")
    base = wid * _QPW
    pltpu.sync_copy(cand_hbm.at[pl.ds(wid * _QPW * _CAND, _QPW * _CAND)],
                    slab_v)
    inf = jnp.full((16,), jnp.inf, dtype=jnp.float32)

    def rbody(r, carry):
        for u in range(4):
            new = []
            for h in range(2):
                ts = carry[h * _K:(h + 1) * _K]
                a = slab_v[pl.ds((4 * r + u) * _QPW + h * 16, 16)]
                for t in ts:
                    new.append(jnp.minimum(t, a))
                    a = jnp.maximum(t, a)
            carry = tuple(new)
        return carry

    tops = jax.lax.fori_loop(0, _CAND // 4, rbody, (inf,) * (2 * _K))
    for h in range(2):
        for i, t in enumerate(tops[h * _K:(h + 1) * _K]):
            out_v[pl.ds((i * 2 + h) * 16, 16)] = t
    pltpu.sync_copy(out_v, out_hbm.at[pl.ds(base * _K, _QPW * _K)])


def _sc_merge(cand_flat):
    mesh = plsc.VectorSubcoreMesh(core_axis_name="c", subcore_axis_name="s")
    f = functools.partial(
        pl.kernel,
        mesh=mesh,
        out_type=jax.ShapeDtypeStruct((_Q * _K,), jnp.float32),
        scratch_types=[
            pltpu.VMEM((_QPW * _CAND,), jnp.float32),
            pltpu.VMEM((_QPW * _K,), jnp.float32),
        ],
    )(_sc_vote_body)
    return f(cand_flat)


@jax.jit
def kernel(x, X_train, y_train):
    n = X_train.shape[0]
    nc = (n + _C - 1) // _C
    npad = nc * _C - n
    m_and = jnp.pad(jnp.full((1, n), -2, dtype=jnp.int32), ((0, 0), (0, npad)))
    m_or = jnp.pad(y_train[None, :], ((0, 0), (0, npad)),
                   constant_values=0x7F7FFFFF)
    tqp = jnp.pad(jnp.sum(X_train * X_train, axis=1)[None, :],
                  ((0, 0), (0, npad)))
    x2 = x.reshape(_Q, _D) * jnp.float32(-2.0)

    out = pl.pallas_call(
        functools.partial(_knn_body, nsteps=nc),
        grid=(nc,),
        in_specs=[
            pl.BlockSpec((_Q, _D), lambda j: (0, 0)),
            pl.BlockSpec((_C, _D), lambda j: (j, 0)),
            pl.BlockSpec((1, _C), lambda j: (0, j)),
            pl.BlockSpec((1, _C), lambda j: (0, j)),
            pl.BlockSpec((1, _C), lambda j: (0, j)),
        ],
        out_specs=pl.BlockSpec((_Q, 512), lambda j: (0, 0)),
        out_shape=jax.ShapeDtypeStruct((_Q, 512), jnp.float32),
        scratch_shapes=[pltpu.VMEM((_Q, 128), jnp.float32)] * 4,
    )(x2, X_train, m_and, m_or, tqp)
    candw = out.reshape(_NW, _QPW, 512).transpose(0, 2, 1)
    top5 = _sc_merge(candw.reshape(-1))
    bits = jnp.bitwise_and(
        jax.lax.bitcast_convert_type(top5, jnp.int32), jnp.int32(1))
    votes = jnp.sum(bits.reshape(_NW, _K, _QPW), axis=1).reshape(_Q, 1)
    return (votes > _K // 2).astype(jnp.float32)
